# baseline (speedup 1.0000x reference)
import functools

import jax
import jax.numpy as jnp
from jax import lax
from jax.experimental import pallas as pl
from jax.experimental.pallas import tpu as pltpu
from jax.experimental.pallas import tpu_sc as plsc

N_NODES = 10000
N_EDGES = 320000
D = 128

NC = 2
NS = 16
NW = NC * NS

NPT = 320
NPT_LAST = N_NODES - (NW - 1) * NPT
NPS = NS * NPT
ACC_ROWS = NPS + 8
PAD_ROW = NPS
CHUNK = 128
NBUF = 5
NBND = 48


def _sc_segment_sum(edges, idx, bounds):
    mesh = plsc.VectorSubcoreMesh(core_axis_name="c", subcore_axis_name="s")

    @functools.partial(
        pl.kernel,
        out_type=jax.ShapeDtypeStruct((N_NODES, D), jnp.float32),
        mesh=mesh,
        scratch_types=dict(
            sacc=pltpu.VMEM_SHARED((ACC_ROWS, D), jnp.float32),
            bvec=pltpu.VMEM((NBND,), jnp.int32),
            bsm=pltpu.SMEM((2,), jnp.int32),
            iraw=pltpu.VMEM((NBUF, CHUNK), jnp.int32),
            iloc=pltpu.VMEM((NBUF, 1, CHUNK), jnp.int32),
            ebuf=pltpu.VMEM((NBUF, CHUNK, D), jnp.float32),
            sems=pltpu.SemaphoreType.DMA((NBUF,)),
        ),
    )
    def k(edges_hbm, idx_hbm, bounds_hbm, out_hbm, sacc, bvec, bsm, iraw,
          iloc, ebuf, sems):
        core = lax.axis_index("c")
        sub = lax.axis_index("s")
        wid = core * NS + sub

        def zero_row(r, carry):
            for j in range(D // 16):
                ebuf[0, r, pl.ds(j * 16, 16)] = jnp.zeros((16,), jnp.float32)
            return carry

        lax.fori_loop(0, CHUNK, zero_row, 0)
        for kk in range(NPT // CHUNK):
            pltpu.sync_copy(
                ebuf.at[0],
                sacc.at[pl.ds(sub * NPT + kk * CHUNK, CHUNK)],
            )
        if NPT % CHUNK:
            pltpu.sync_copy(
                ebuf.at[0, pl.ds(0, NPT % CHUNK)],
                sacc.at[pl.ds(sub * NPT + (NPT // CHUNK) * CHUNK, NPT % CHUNK)],
            )

        @pl.when(sub == NS - 1)
        def _():
            pltpu.sync_copy(ebuf.at[0, pl.ds(0, 8)], sacc.at[pl.ds(NPS, 8)])

        plsc.subcore_barrier()

        pltpu.sync_copy(bounds_hbm, bvec)
        lanes = lax.iota(jnp.int32, 16)

        for w in range(NW):
            @pl.when(wid == w)
            def _(w=w):
                q0, l0 = divmod(w, 16)
                q1, l1 = divmod(w + 1, 16)
                v0 = bvec[pl.ds(q0 * 16, 16)]
                bsm[0] = lax.squeeze(lax.slice(v0, (l0,), (l0 + 1,)), (0,))
                v1 = bvec[pl.ds(q1 * 16, 16)]
                bsm[1] = lax.squeeze(lax.slice(v1, (l1,), (l1 + 1,)), (0,))

        estart = bsm[0] * 8
        eend = bsm[1] * 8 + 8
        nch = (eend - estart + (CHUNK - 1)) // CHUNK
        node_lo = wid * NPT
        node_hi = jnp.minimum((wid + 1) * NPT, N_NODES)
        sc_base = core * NPS

        def chunk_start(c):
            raw = estart + c * CHUNK
            return jnp.minimum(raw, N_EDGES - CHUNK), raw

        def start_gather(c, b):
            sc, _ = chunk_start(c)
            sc = pl.multiple_of(sc, 8)
            pltpu.async_copy(edges_hbm.at[pl.ds(sc, CHUNK)], ebuf.at[b], sems.at[b])
            pltpu.async_copy(idx_hbm.at[pl.ds(sc, CHUNK)], iraw.at[b], sems.at[b])

        def wait_gather(b):
            pltpu.make_async_copy(
                edges_hbm.at[pl.ds(0, CHUNK)], ebuf.at[b], sems.at[b]
            ).wait()
            pltpu.make_async_copy(
                idx_hbm.at[pl.ds(0, CHUNK)], iraw.at[b], sems.at[b]
            ).wait()

        def process_chunk(c, b):
            sc, raw = chunk_start(c)
            for q in range(CHUNK // 16):
                v = iraw[b, pl.ds(q * 16, 16)]
                e = sc + q * 16 + lanes
                keep = (v >= node_lo) & (v < node_hi) & (e >= raw)
                iloc[b, 0, pl.ds(q * 16, 16)] = jnp.where(
                    keep, v - sc_base, jnp.int32(PAD_ROW)
                )
            pltpu.sync_copy(ebuf.at[b], sacc.at[iloc.at[b, 0]], add=True)

        for b in range(NBUF):
            @pl.when(b < nch)
            def _(b=b):
                start_gather(b, b)

        def body(g, carry):
            for b in range(NBUF):
                c = g * NBUF + b

                @pl.when(c < nch)
                def _(c=c, b=b):
                    wait_gather(b)
                    process_chunk(c, b)

                    @pl.when(c + NBUF < nch)
                    def _():
                        start_gather(c + NBUF, b)

            return carry

        lax.fori_loop(0, (nch + NBUF - 1) // NBUF, body, 0)
        plsc.subcore_barrier()

        @pl.when(wid < NW - 1)
        def _():
            pltpu.sync_copy(
                sacc.at[pl.ds(sub * NPT, NPT)],
                out_hbm.at[pl.ds(wid * NPT, NPT)],
            )

        @pl.when(wid == NW - 1)
        def _():
            pltpu.sync_copy(
                sacc.at[pl.ds(sub * NPT, NPT_LAST)],
                out_hbm.at[pl.ds((NW - 1) * NPT, NPT_LAST)],
            )

    return k(edges, idx, bounds)


@jax.jit
def kernel(new_edges, recv_idx):
    idx = recv_idx.astype(jnp.int32)
    starts = jnp.arange(NBND, dtype=jnp.int32) * NPT
    grp = jnp.sum(
        (idx[7::8][:, None] < starts[None, :]).astype(jnp.int32), axis=0
    )
    return _sc_segment_sum(new_edges, idx, grp)


# speedup vs baseline: 1.0248x; 1.0248x over previous
import functools

import jax
import jax.numpy as jnp
from jax import lax
from jax.experimental import pallas as pl
from jax.experimental.pallas import tpu as pltpu
from jax.experimental.pallas import tpu_sc as plsc

N_NODES = 10000
N_EDGES = 320000
D = 128

NC = 2
NS = 16
NW = NC * NS

NPT = 320
NPT_LAST = N_NODES - (NW - 1) * NPT
NPS = NS * NPT
ACC_ROWS = NPS + 8
PAD_ROW = NPS
CHUNK = 128
NBUF = 5
NBND = 48


def _sc_segment_sum(edges, idx, bounds):
    mesh = plsc.VectorSubcoreMesh(core_axis_name="c", subcore_axis_name="s")

    @functools.partial(
        pl.kernel,
        out_type=jax.ShapeDtypeStruct((N_NODES, D), jnp.float32),
        mesh=mesh,
        scratch_types=dict(
            sacc=pltpu.VMEM_SHARED((ACC_ROWS, D), jnp.float32),
            bvec=pltpu.VMEM((NBND,), jnp.int32),
            bsm=pltpu.SMEM((2,), jnp.int32),
            iraw=pltpu.VMEM((NBUF, CHUNK), jnp.int32),
            iloc=pltpu.VMEM((NBUF, 1, CHUNK), jnp.int32),
            ebuf=pltpu.VMEM((NBUF, CHUNK, D), jnp.float32),
            sems=pltpu.SemaphoreType.DMA((NBUF,)),
        ),
    )
    def k(edges_hbm, idx_hbm, bounds_hbm, out_hbm, sacc, bvec, bsm, iraw,
          iloc, ebuf, sems):
        core = lax.axis_index("c")
        sub = lax.axis_index("s")
        wid = core * NS + sub

        pltpu.sync_copy(bounds_hbm, bvec)
        lanes = lax.iota(jnp.int32, 16)

        for w in range(NW):
            @pl.when(wid == w)
            def _(w=w):
                q0, l0 = divmod(w, 16)
                q1, l1 = divmod(w + 1, 16)
                v0 = bvec[pl.ds(q0 * 16, 16)]
                bsm[0] = lax.squeeze(lax.slice(v0, (l0,), (l0 + 1,)), (0,))
                v1 = bvec[pl.ds(q1 * 16, 16)]
                bsm[1] = lax.squeeze(lax.slice(v1, (l1,), (l1 + 1,)), (0,))

        b_lo = bsm[0]
        b_hi = bsm[1]
        estart = (b_lo // 8) * 8
        eend = ((b_hi + 7) // 8) * 8
        nch = (eend - estart + (CHUNK - 1)) // CHUNK
        node_lo = wid * NPT
        node_hi = jnp.minimum((wid + 1) * NPT, N_NODES)
        sc_base = core * NPS

        def chunk_start(c):
            raw = estart + c * CHUNK
            return jnp.minimum(raw, N_EDGES - CHUNK), raw

        def start_gather(c, b):
            sc, _ = chunk_start(c)
            sc = pl.multiple_of(sc, 8)
            pltpu.async_copy(edges_hbm.at[pl.ds(sc, CHUNK)], ebuf.at[b], sems.at[b])
            pltpu.async_copy(idx_hbm.at[pl.ds(sc, CHUNK)], iraw.at[b], sems.at[b])

        def wait_gather(b):
            pltpu.make_async_copy(
                edges_hbm.at[pl.ds(0, CHUNK)], ebuf.at[b], sems.at[b]
            ).wait()
            pltpu.make_async_copy(
                idx_hbm.at[pl.ds(0, CHUNK)], iraw.at[b], sems.at[b]
            ).wait()

        def process_chunk(c, b):
            sc, raw = chunk_start(c)
            for q in range(CHUNK // 16):
                v = iraw[b, pl.ds(q * 16, 16)]
                e = sc + q * 16 + lanes
                keep = (v >= node_lo) & (v < node_hi) & (e >= raw)
                iloc[b, 0, pl.ds(q * 16, 16)] = jnp.where(
                    keep, v - sc_base, jnp.int32(PAD_ROW)
                )
            pltpu.sync_copy(ebuf.at[b], sacc.at[iloc.at[b, 0]], add=True)

        for b in range(1, NBUF):
            @pl.when(b < nch)
            def _(b=b):
                start_gather(b, b)

        def zero_row(r, carry):
            for j in range(D // 16):
                ebuf[0, r, pl.ds(j * 16, 16)] = jnp.zeros((16,), jnp.float32)
            return carry

        lax.fori_loop(0, CHUNK, zero_row, 0)
        for kk in range(NPT // CHUNK):
            pltpu.sync_copy(
                ebuf.at[0],
                sacc.at[pl.ds(sub * NPT + kk * CHUNK, CHUNK)],
            )
        if NPT % CHUNK:
            pltpu.sync_copy(
                ebuf.at[0, pl.ds(0, NPT % CHUNK)],
                sacc.at[pl.ds(sub * NPT + (NPT // CHUNK) * CHUNK, NPT % CHUNK)],
            )

        @pl.when(sub == NS - 1)
        def _():
            pltpu.sync_copy(ebuf.at[0, pl.ds(0, 8)], sacc.at[pl.ds(NPS, 8)])

        plsc.subcore_barrier()

        @pl.when(0 < nch)
        def _():
            start_gather(0, 0)

        def body(g, carry):
            for b in range(NBUF):
                c = g * NBUF + b

                @pl.when(c < nch)
                def _(c=c, b=b):
                    wait_gather(b)
                    process_chunk(c, b)

                    @pl.when(c + NBUF < nch)
                    def _():
                        start_gather(c + NBUF, b)

            return carry

        lax.fori_loop(0, (nch + NBUF - 1) // NBUF, body, 0)
        plsc.subcore_barrier()

        @pl.when(wid < NW - 1)
        def _():
            pltpu.sync_copy(
                sacc.at[pl.ds(sub * NPT, NPT)],
                out_hbm.at[pl.ds(wid * NPT, NPT)],
            )

        @pl.when(wid == NW - 1)
        def _():
            pltpu.sync_copy(
                sacc.at[pl.ds(sub * NPT, NPT_LAST)],
                out_hbm.at[pl.ds((NW - 1) * NPT, NPT_LAST)],
            )

    return k(edges, idx, bounds)


@jax.jit
def kernel(new_edges, recv_idx):
    idx = recv_idx.astype(jnp.int32)
    starts = jnp.arange(NBND, dtype=jnp.int32) * NPT
    bounds = jnp.sum(
        (idx[:, None] < starts[None, :]).astype(jnp.int32), axis=0
    )
    return _sc_segment_sum(new_edges, idx, bounds)
